# Initial kernel scaffold; baseline (speedup 1.0000x reference)
import jax
import jax.numpy as jnp
from jax.experimental import pallas as pl


def kernel(x, edge_index, fc_W, fc_b, W1, b1, W2, b2):
    raise NotImplementedError("write your pallas kernel here")


# speedup vs baseline: 29.6800x; 29.6800x over previous
import functools

import jax
import jax.numpy as jnp
from jax import lax
from jax.experimental import pallas as pl
from jax.experimental.pallas import tpu as pltpu
from jax.experimental.pallas import tpu_sc as plsc

N = 100000
E = 1600000
D_IN = 128
D_LAT = 32
D_OUT = 128
HALF = 16

NSUB = 16
SHARD = E // NSUB
SUB = 128
CHUNK = 1024
NSUBD = CHUNK // SUB
PAD = (-SHARD) % CHUNK
SHARD_P = SHARD + PAD
NCHUNK = SHARD_P // CHUNK
IDX_ROWS = SHARD_P // SUB
ACC_ROWS = N + PAD
ZROWS = ACC_ROWS // NSUB
OUT_ROWS = 6256
OUT_TAIL = N - 15 * OUT_ROWS

BLK = 2000
GRID = N // BLK

_mesh = plsc.VectorSubcoreMesh(core_axis_name="c", subcore_axis_name="s")
_sc_params = pltpu.CompilerParams(use_tc_tiling_on_sc=False,
                                  internal_scratch_in_bytes=0)


def _zero_fill(buf, nrows):
    zero = jnp.zeros((16,), jnp.float32)

    def zb(i, carry):
        buf[i] = zero
        return carry

    lax.fori_loop(0, nrows, zb, 0)


def _sc_conv_body(srcp, dstp, g_lo, g_hi, out_lo, out_hi,
                  srcv, dstv, rows, acc, isem, gsem, ssem):
    c = lax.axis_index("c")
    t = lax.axis_index("s")

    _zero_fill(rows, CHUNK)
    zbase = t * ZROWS
    for j in range(ZROWS // CHUNK):
        pltpu.sync_copy(rows.at[pl.ds(0, CHUNK)],
                        acc.at[pl.ds(zbase + j * CHUNK, CHUNK)])
    rem = ZROWS % CHUNK
    if rem:
        pltpu.sync_copy(rows.at[pl.ds(0, rem)],
                        acc.at[pl.ds(zbase + (ZROWS // CHUNK) * CHUNK, rem)])
    plsc.subcore_barrier()

    def run_chunks(g_hbm):
        def chunk(m, carry):
            ib = t * IDX_ROWS + m * NSUBD
            cs = pltpu.async_copy(srcp.at[pl.ds(ib, NSUBD)], srcv, isem)
            cd = pltpu.async_copy(dstp.at[pl.ds(ib, NSUBD)], dstv, isem)
            cs.wait()
            cd.wait()
            hs = [pltpu.async_copy(g_hbm.at[srcv.at[j]],
                                   rows.at[pl.ds(j * SUB, SUB)], gsem)
                  for j in range(NSUBD)]
            for h in hs:
                h.wait()
            ss = [pltpu.async_copy(rows.at[pl.ds(j * SUB, SUB)],
                                   acc.at[dstv.at[j]], ssem, add=True)
                  for j in range(NSUBD)]
            for s in ss:
                s.wait()
            return carry

        lax.fori_loop(0, NCHUNK, chunk, 0)

    pl.when(c == 0)(lambda: run_chunks(g_lo))
    pl.when(c == 1)(lambda: run_chunks(g_hi))
    plsc.subcore_barrier()

    _write_out(c, t, acc, out_lo, out_hi)


def _write_out(c, t, acc, out_lo, out_hi):
    ob = pl.multiple_of(t * OUT_ROWS, 8)

    def wr(out, base, nrows):
        pltpu.sync_copy(acc.at[pl.ds(base, nrows)], out.at[pl.ds(base, nrows)])

    pl.when((c == 0) & (t < 15))(lambda: wr(out_lo, ob, OUT_ROWS))
    pl.when((c == 1) & (t < 15))(lambda: wr(out_hi, ob, OUT_ROWS))
    pl.when((c == 0) & (t == 15))(lambda: wr(out_lo, 15 * OUT_ROWS, OUT_TAIL))
    pl.when((c == 1) & (t == 15))(lambda: wr(out_hi, 15 * OUT_ROWS, OUT_TAIL))


_sc_conv = pl.kernel(
    _sc_conv_body,
    out_type=[jax.ShapeDtypeStruct((N, HALF), jnp.float32),
              jax.ShapeDtypeStruct((N, HALF), jnp.float32)],
    mesh=_mesh,
    scratch_types=[
        pltpu.VMEM((NSUBD, SUB), jnp.int32),
        pltpu.VMEM((NSUBD, SUB), jnp.int32),
        pltpu.VMEM((CHUNK, HALF), jnp.float32),
        pltpu.VMEM_SHARED((ACC_ROWS, HALF), jnp.float32),
        pltpu.SemaphoreType.DMA,
        pltpu.SemaphoreType.DMA,
        pltpu.SemaphoreType.DMA,
    ],
    compiler_params=_sc_params,
)


def _sc_deg_body(dstp, out_lo, out_hi, dstv, ones, zbuf, acc, isem, ssem):
    c = lax.axis_index("c")
    t = lax.axis_index("s")

    one = jnp.full((16,), 1.0, jnp.float32)

    def ob_(i, carry):
        ones[i] = one
        return carry

    lax.fori_loop(0, SUB, ob_, 0)

    _zero_fill(zbuf, CHUNK)
    zbase = t * ZROWS
    for j in range(ZROWS // CHUNK):
        pltpu.sync_copy(zbuf.at[pl.ds(0, CHUNK)],
                        acc.at[pl.ds(zbase + j * CHUNK, CHUNK)])
    rem = ZROWS % CHUNK
    if rem:
        pltpu.sync_copy(zbuf.at[pl.ds(0, rem)],
                        acc.at[pl.ds(zbase + (ZROWS // CHUNK) * CHUNK, rem)])
    plsc.subcore_barrier()

    def chunk(i, carry):
        m = 2 * i + c

        @pl.when(m < NCHUNK)
        def _():
            ib = t * IDX_ROWS + m * NSUBD
            pltpu.async_copy(dstp.at[pl.ds(ib, NSUBD)], dstv, isem).wait()
            ss = [pltpu.async_copy(ones, acc.at[dstv.at[j]], ssem, add=True)
                  for j in range(NSUBD)]
            for s in ss:
                s.wait()

        return carry

    lax.fori_loop(0, (NCHUNK + 1) // 2, chunk, 0)
    plsc.subcore_barrier()

    _write_out(c, t, acc, out_lo, out_hi)


_sc_deg = pl.kernel(
    _sc_deg_body,
    out_type=[jax.ShapeDtypeStruct((N, HALF), jnp.float32),
              jax.ShapeDtypeStruct((N, HALF), jnp.float32)],
    mesh=_mesh,
    scratch_types=[
        pltpu.VMEM((NSUBD, SUB), jnp.int32),
        pltpu.VMEM((SUB, HALF), jnp.float32),
        pltpu.VMEM((CHUNK, HALF), jnp.float32),
        pltpu.VMEM_SHARED((ACC_ROWS, HALF), jnp.float32),
        pltpu.SemaphoreType.DMA,
        pltpu.SemaphoreType.DMA,
    ],
    compiler_params=_sc_params,
)


def _stage_a_body(x_ref, w_ref, b_ref, da_ref, db_ref,
                  dinv_ref, gl_ref, gr_ref):
    h = jnp.dot(x_ref[...], w_ref[...], preferred_element_type=jnp.float32)
    h = jnp.maximum(h + b_ref[...], 0.0)
    deg = da_ref[:, :1] + db_ref[:, :1] + 1.0
    dinv = lax.rsqrt(deg)
    g = h * dinv
    dinv_ref[...] = jnp.broadcast_to(dinv, (BLK, HALF))
    gl_ref[...] = g[:, :HALF]
    gr_ref[...] = g[:, HALF:]


_stage_a = pl.pallas_call(
    _stage_a_body,
    grid=(GRID,),
    in_specs=[
        pl.BlockSpec((BLK, D_IN), lambda i: (i, 0)),
        pl.BlockSpec((D_IN, D_LAT), lambda i: (0, 0)),
        pl.BlockSpec((1, D_LAT), lambda i: (0, 0)),
        pl.BlockSpec((BLK, HALF), lambda i: (i, 0)),
        pl.BlockSpec((BLK, HALF), lambda i: (i, 0)),
    ],
    out_specs=[
        pl.BlockSpec((BLK, HALF), lambda i: (i, 0)),
        pl.BlockSpec((BLK, HALF), lambda i: (i, 0)),
        pl.BlockSpec((BLK, HALF), lambda i: (i, 0)),
    ],
    out_shape=[jax.ShapeDtypeStruct((N, HALF), jnp.float32)] * 3,
)


def _stage_b_body(al_ref, ar_ref, gl_ref, gr_ref, dinv_ref, w_ref, b_ref,
                  ol_ref, or_ref):
    dinv = dinv_ref[:, :1]
    s = jnp.concatenate([al_ref[...] + gl_ref[...],
                         ar_ref[...] + gr_ref[...]], axis=1) * dinv
    h = jnp.dot(s, w_ref[...], preferred_element_type=jnp.float32)
    h = jnp.maximum(h + b_ref[...], 0.0)
    g2 = h * dinv
    ol_ref[...] = g2[:, :HALF]
    or_ref[...] = g2[:, HALF:]


_stage_b = pl.pallas_call(
    _stage_b_body,
    grid=(GRID,),
    in_specs=[
        pl.BlockSpec((BLK, HALF), lambda i: (i, 0)),
        pl.BlockSpec((BLK, HALF), lambda i: (i, 0)),
        pl.BlockSpec((BLK, HALF), lambda i: (i, 0)),
        pl.BlockSpec((BLK, HALF), lambda i: (i, 0)),
        pl.BlockSpec((BLK, HALF), lambda i: (i, 0)),
        pl.BlockSpec((D_LAT, D_LAT), lambda i: (0, 0)),
        pl.BlockSpec((1, D_LAT), lambda i: (0, 0)),
    ],
    out_specs=[
        pl.BlockSpec((BLK, HALF), lambda i: (i, 0)),
        pl.BlockSpec((BLK, HALF), lambda i: (i, 0)),
    ],
    out_shape=[jax.ShapeDtypeStruct((N, HALF), jnp.float32)] * 2,
)


def _stage_c_body(al_ref, ar_ref, gl_ref, gr_ref, dinv_ref, w_ref, b_ref,
                  o_ref):
    dinv = dinv_ref[:, :1]
    s = jnp.concatenate([al_ref[...] + gl_ref[...],
                         ar_ref[...] + gr_ref[...]], axis=1) * dinv
    o_ref[...] = jnp.dot(s, w_ref[...],
                         preferred_element_type=jnp.float32) + b_ref[...]


_stage_c = pl.pallas_call(
    _stage_c_body,
    grid=(GRID,),
    in_specs=[
        pl.BlockSpec((BLK, HALF), lambda i: (i, 0)),
        pl.BlockSpec((BLK, HALF), lambda i: (i, 0)),
        pl.BlockSpec((BLK, HALF), lambda i: (i, 0)),
        pl.BlockSpec((BLK, HALF), lambda i: (i, 0)),
        pl.BlockSpec((BLK, HALF), lambda i: (i, 0)),
        pl.BlockSpec((D_LAT, D_OUT), lambda i: (0, 0)),
        pl.BlockSpec((1, D_OUT), lambda i: (0, 0)),
    ],
    out_specs=pl.BlockSpec((BLK, D_OUT), lambda i: (i, 0)),
    out_shape=jax.ShapeDtypeStruct((N, D_OUT), jnp.float32),
)


def kernel(x, edge_index, fc_W, fc_b, W1, b1, W2, b2):
    src = edge_index[0]
    dst = edge_index[1]
    pad = jnp.arange(PAD, dtype=jnp.int32)
    srcp = jnp.concatenate(
        [src.reshape(NSUB, SHARD), jnp.broadcast_to(pad, (NSUB, PAD))],
        axis=1).reshape(NSUB * IDX_ROWS, SUB)
    dstp = jnp.concatenate(
        [dst.reshape(NSUB, SHARD), jnp.broadcast_to(pad + N, (NSUB, PAD))],
        axis=1).reshape(NSUB * IDX_ROWS, SUB)

    deg_lo, deg_hi = _sc_deg(dstp)
    dinv, g1l, g1r = _stage_a(x, fc_W, fc_b.reshape(1, -1), deg_lo, deg_hi)
    a1l, a1r = _sc_conv(srcp, dstp, g1l, g1r)
    g2l, g2r = _stage_b(a1l, a1r, g1l, g1r, dinv, W1, b1.reshape(1, -1))
    a2l, a2r = _sc_conv(srcp, dstp, g2l, g2r)
    out = _stage_c(a2l, a2r, g2l, g2r, dinv, W2, b2.reshape(1, -1))
    return out


# speedup vs baseline: 49.2088x; 1.6580x over previous
import jax
import jax.numpy as jnp
from jax import lax
from jax.experimental import pallas as pl
from jax.experimental.pallas import tpu as pltpu
from jax.experimental.pallas import tpu_sc as plsc

N = 100000
E = 1600000
D_IN = 128
D_LAT = 32
D_OUT = 128
HALF = 16

NSUB = 16
SUB = 128
CHUNK = 512
NSUBD = CHUNK // SUB
IDXR = E // SUB
TOTCH = E // CHUNK
NCHT = -(-TOTCH // NSUB)
ZTAIL = (N // NSUB) % CHUNK
OUT_ROWS = N // NSUB

PB = 512
PROWS = N * HALF // 128
GRID = -(-PROWS // PB)

_mesh = plsc.VectorSubcoreMesh(core_axis_name="c", subcore_axis_name="s")
_sc_params = pltpu.CompilerParams(use_tc_tiling_on_sc=False)


def _zero_fill(buf, nrows):
    zero = jnp.zeros((16,), jnp.float32)

    def zb(i, carry):
        buf[i] = zero
        return carry

    lax.fori_loop(0, nrows, zb, 0)


def _zero_acc(t, zbuf, acc):
    zbase = t * OUT_ROWS
    for j in range(OUT_ROWS // CHUNK):
        pltpu.sync_copy(zbuf.at[pl.ds(0, CHUNK)],
                        acc.at[pl.ds(zbase + j * CHUNK, CHUNK)])
    if ZTAIL:
        pltpu.sync_copy(zbuf.at[pl.ds(0, ZTAIL)],
                        acc.at[pl.ds(zbase + (OUT_ROWS // CHUNK) * CHUNK,
                                     ZTAIL)])


def _write_out(c, t, acc, out_lo, out_hi):
    ob = t * OUT_ROWS
    pl.when(c == 0)(lambda: pltpu.sync_copy(acc.at[pl.ds(ob, OUT_ROWS)],
                                            out_lo.at[pl.ds(ob, OUT_ROWS)]))
    pl.when(c == 1)(lambda: pltpu.sync_copy(acc.at[pl.ds(ob, OUT_ROWS)],
                                            out_hi.at[pl.ds(ob, OUT_ROWS)]))


def _sc_conv_body(srcp, dstp, g_lo, g_hi, out_lo, out_hi,
                  srcv, dstv, rows0, rows1, acc,
                  isem0, isem1, isem2, isem3, gsem, ssem0, ssem1):
    c = lax.axis_index("c")
    t = lax.axis_index("s")
    rows = (rows0, rows1)
    isems = (isem0, isem1, isem2, isem3)
    ssems = (ssem0, ssem1)

    _zero_fill(rows0, CHUNK)
    _zero_acc(t, rows0, acc)

    for q in range(2):
        m = 16 * q + t
        pltpu.async_copy(srcp.at[pl.ds(m * NSUBD, NSUBD)], srcv.at[q],
                         isems[q])
        pltpu.async_copy(dstp.at[pl.ds(m * NSUBD, NSUBD)], dstv.at[q],
                         isems[q])
    plsc.subcore_barrier()

    def run_chunks(g_hbm):
        def chunk_iter(i, carry):
            for q in range(4):
                j = 4 * i + q
                m = 16 * j + t

                @pl.when(m < TOTCH)
                def _(j=j, m=m, q=q):
                    h = q % 2
                    if q >= 2:
                        pltpu.make_async_copy(
                            g_hbm.at[pl.ds(0, CHUNK)], rows[h],
                            ssems[h]).wait()
                    else:
                        @pl.when(j >= 2)
                        def _():
                            pltpu.make_async_copy(
                                g_hbm.at[pl.ds(0, CHUNK)], rows[h],
                                ssems[h]).wait()
                    mn = m + 32
                    qn = (q + 2) % 4

                    @pl.when(mn < TOTCH)
                    def _(mn=mn, qn=qn):
                        pltpu.async_copy(srcp.at[pl.ds(mn * NSUBD, NSUBD)],
                                         srcv.at[qn], isems[qn])
                        pltpu.async_copy(dstp.at[pl.ds(mn * NSUBD, NSUBD)],
                                         dstv.at[qn], isems[qn])
                    pltpu.make_async_copy(srcp.at[pl.ds(0, NSUBD)],
                                          srcv.at[q], isems[q]).wait()
                    pltpu.make_async_copy(dstp.at[pl.ds(0, NSUBD)],
                                          dstv.at[q], isems[q]).wait()
                    hs = [pltpu.async_copy(
                        g_hbm.at[srcv.at[q, jj]],
                        rows[h].at[pl.ds(jj * SUB, SUB)], gsem)
                        for jj in range(NSUBD)]
                    for hh in hs:
                        hh.wait()
                    for jj in range(NSUBD):
                        pltpu.async_copy(rows[h].at[pl.ds(jj * SUB, SUB)],
                                         acc.at[dstv.at[q, jj]], ssems[h],
                                         add=True)
            return carry

        lax.fori_loop(0, NCHT // 4, chunk_iter, 0)
        pltpu.make_async_copy(g_hbm.at[pl.ds(0, CHUNK)], rows[0],
                              ssems[0]).wait()
        pltpu.make_async_copy(g_hbm.at[pl.ds(0, CHUNK)], rows[1],
                              ssems[1]).wait()

    pl.when(c == 0)(lambda: run_chunks(g_lo))
    pl.when(c == 1)(lambda: run_chunks(g_hi))
    plsc.subcore_barrier()
    _write_out(c, t, acc, out_lo, out_hi)


_sc_conv = pl.kernel(
    _sc_conv_body,
    out_type=[jax.ShapeDtypeStruct((N, HALF), jnp.float32),
              jax.ShapeDtypeStruct((N, HALF), jnp.float32)],
    mesh=_mesh,
    scratch_types=[
        pltpu.VMEM((4, NSUBD, SUB), jnp.int32),
        pltpu.VMEM((4, NSUBD, SUB), jnp.int32),
        pltpu.VMEM((CHUNK, HALF), jnp.float32),
        pltpu.VMEM((CHUNK, HALF), jnp.float32),
        pltpu.VMEM_SHARED((N, HALF), jnp.float32),
        pltpu.SemaphoreType.DMA,
        pltpu.SemaphoreType.DMA,
        pltpu.SemaphoreType.DMA,
        pltpu.SemaphoreType.DMA,
        pltpu.SemaphoreType.DMA,
        pltpu.SemaphoreType.DMA,
        pltpu.SemaphoreType.DMA,
    ],
    compiler_params=_sc_params,
)


def _sc_deg_body(dstp, out_lo, out_hi, dstv, ones, acc, isem, ssem):
    c = lax.axis_index("c")
    t = lax.axis_index("s")
    w = c * NSUB + t

    _zero_fill(ones, CHUNK)
    _zero_acc(t, ones, acc)

    one = jnp.full((16,), 1.0, jnp.float32)

    def ob_(i, carry):
        ones[i] = one
        return carry

    lax.fori_loop(0, SUB, ob_, 0)
    plsc.subcore_barrier()

    def chunk(i, carry):
        m = 32 * i + w

        @pl.when(m < TOTCH)
        def _():
            pltpu.async_copy(dstp.at[pl.ds(m * NSUBD, NSUBD)], dstv,
                             isem).wait()
            ss = [pltpu.async_copy(ones.at[pl.ds(0, SUB)],
                                   acc.at[dstv.at[jj]], ssem, add=True)
                  for jj in range(NSUBD)]
            for s in ss:
                s.wait()
        return carry

    lax.fori_loop(0, -(-TOTCH // 32), chunk, 0)
    plsc.subcore_barrier()
    _write_out(c, t, acc, out_lo, out_hi)


_sc_deg = pl.kernel(
    _sc_deg_body,
    out_type=[jax.ShapeDtypeStruct((N, HALF), jnp.float32),
              jax.ShapeDtypeStruct((N, HALF), jnp.float32)],
    mesh=_mesh,
    scratch_types=[
        pltpu.VMEM((NSUBD, SUB), jnp.int32),
        pltpu.VMEM((CHUNK, HALF), jnp.float32),
        pltpu.VMEM_SHARED((N, HALF), jnp.float32),
        pltpu.SemaphoreType.DMA,
        pltpu.SemaphoreType.DMA,
    ],
    compiler_params=_sc_params,
)


def _stage_a_body(xp_ref, bdl_ref, bdh_ref, bl_ref, bh_ref, dl_ref, dh_ref,
                  dinv_ref, gl_ref, gr_ref):
    deg = dl_ref[...] + dh_ref[...] + 1.0
    dinv = lax.rsqrt(deg)
    hl = jnp.dot(xp_ref[...], bdl_ref[...],
                 preferred_element_type=jnp.float32) + bl_ref[...]
    hr = jnp.dot(xp_ref[...], bdh_ref[...],
                 preferred_element_type=jnp.float32) + bh_ref[...]
    dinv_ref[...] = dinv
    gl_ref[...] = jnp.maximum(hl, 0.0) * dinv
    gr_ref[...] = jnp.maximum(hr, 0.0) * dinv


_stage_a = pl.pallas_call(
    _stage_a_body,
    grid=(GRID,),
    in_specs=[
        pl.BlockSpec((PB, 8 * D_IN), lambda i: (i, 0)),
        pl.BlockSpec((8 * D_IN, 128), lambda i: (0, 0)),
        pl.BlockSpec((8 * D_IN, 128), lambda i: (0, 0)),
        pl.BlockSpec((1, 128), lambda i: (0, 0)),
        pl.BlockSpec((1, 128), lambda i: (0, 0)),
        pl.BlockSpec((PB, 128), lambda i: (i, 0)),
        pl.BlockSpec((PB, 128), lambda i: (i, 0)),
    ],
    out_specs=[
        pl.BlockSpec((PB, 128), lambda i: (i, 0)),
        pl.BlockSpec((PB, 128), lambda i: (i, 0)),
        pl.BlockSpec((PB, 128), lambda i: (i, 0)),
    ],
    out_shape=[jax.ShapeDtypeStruct((PROWS, 128), jnp.float32)] * 3,
)


def _stage_b_body(al_ref, ar_ref, gl_ref, gr_ref, dinv_ref,
                  bll_ref, blh_ref, bhl_ref, bhh_ref, b1l_ref, b1h_ref,
                  ol_ref, or_ref):
    dinv = dinv_ref[...]
    sl = (al_ref[...] + gl_ref[...]) * dinv
    sr = (ar_ref[...] + gr_ref[...]) * dinv
    hl = (jnp.dot(sl, bll_ref[...], preferred_element_type=jnp.float32)
          + jnp.dot(sr, bhl_ref[...], preferred_element_type=jnp.float32)
          + b1l_ref[...])
    hr = (jnp.dot(sl, blh_ref[...], preferred_element_type=jnp.float32)
          + jnp.dot(sr, bhh_ref[...], preferred_element_type=jnp.float32)
          + b1h_ref[...])
    ol_ref[...] = jnp.maximum(hl, 0.0) * dinv
    or_ref[...] = jnp.maximum(hr, 0.0) * dinv


_stage_b = pl.pallas_call(
    _stage_b_body,
    grid=(GRID,),
    in_specs=[pl.BlockSpec((PB, 128), lambda i: (i, 0))] * 5 + [
        pl.BlockSpec((128, 128), lambda i: (0, 0)),
        pl.BlockSpec((128, 128), lambda i: (0, 0)),
        pl.BlockSpec((128, 128), lambda i: (0, 0)),
        pl.BlockSpec((128, 128), lambda i: (0, 0)),
        pl.BlockSpec((1, 128), lambda i: (0, 0)),
        pl.BlockSpec((1, 128), lambda i: (0, 0)),
    ],
    out_specs=[
        pl.BlockSpec((PB, 128), lambda i: (i, 0)),
        pl.BlockSpec((PB, 128), lambda i: (i, 0)),
    ],
    out_shape=[jax.ShapeDtypeStruct((PROWS, 128), jnp.float32)] * 2,
)


def _stage_c_body(al_ref, ar_ref, gl_ref, gr_ref, dinv_ref,
                  bdl_ref, bdh_ref, b2_ref, o_ref):
    dinv = dinv_ref[...]
    sl = (al_ref[...] + gl_ref[...]) * dinv
    sr = (ar_ref[...] + gr_ref[...]) * dinv
    o_ref[...] = (jnp.dot(sl, bdl_ref[...],
                          preferred_element_type=jnp.float32)
                  + jnp.dot(sr, bdh_ref[...],
                            preferred_element_type=jnp.float32)
                  + b2_ref[...])


_stage_c = pl.pallas_call(
    _stage_c_body,
    grid=(GRID,),
    in_specs=[pl.BlockSpec((PB, 128), lambda i: (i, 0))] * 5 + [
        pl.BlockSpec((128, 8 * D_OUT), lambda i: (0, 0)),
        pl.BlockSpec((128, 8 * D_OUT), lambda i: (0, 0)),
        pl.BlockSpec((1, 8 * D_OUT), lambda i: (0, 0)),
    ],
    out_specs=pl.BlockSpec((PB, 8 * D_OUT), lambda i: (i, 0)),
    out_shape=jax.ShapeDtypeStruct((PROWS, 8 * D_OUT), jnp.float32),
)


def _bd(w):
    return jnp.kron(jnp.eye(8, dtype=w.dtype), w)


def kernel(x, edge_index, fc_W, fc_b, W1, b1, W2, b2):
    srcp = edge_index[0].reshape(IDXR, SUB)
    dstp = edge_index[1].reshape(IDXR, SUB)
    xp = x.reshape(PROWS, 8 * D_IN)

    deg_lo, deg_hi = _sc_deg(dstp)
    dinv_p, g1l_p, g1r_p = _stage_a(
        xp, _bd(fc_W[:, :HALF]), _bd(fc_W[:, HALF:]),
        jnp.tile(fc_b[:HALF], 8).reshape(1, 128),
        jnp.tile(fc_b[HALF:], 8).reshape(1, 128),
        deg_lo.reshape(PROWS, 128), deg_hi.reshape(PROWS, 128))

    a1l, a1r = _sc_conv(srcp, dstp,
                        g1l_p.reshape(N, HALF), g1r_p.reshape(N, HALF))
    g2l_p, g2r_p = _stage_b(
        a1l.reshape(PROWS, 128), a1r.reshape(PROWS, 128), g1l_p, g1r_p,
        dinv_p,
        _bd(W1[:HALF, :HALF]), _bd(W1[:HALF, HALF:]),
        _bd(W1[HALF:, :HALF]), _bd(W1[HALF:, HALF:]),
        jnp.tile(b1[:HALF], 8).reshape(1, 128),
        jnp.tile(b1[HALF:], 8).reshape(1, 128))

    a2l, a2r = _sc_conv(srcp, dstp,
                        g2l_p.reshape(N, HALF), g2r_p.reshape(N, HALF))
    out_p = _stage_c(
        a2l.reshape(PROWS, 128), a2r.reshape(PROWS, 128), g2l_p, g2r_p,
        dinv_p,
        _bd(W2[:HALF, :]), _bd(W2[HALF:, :]),
        jnp.tile(b2, 8).reshape(1, 8 * D_OUT))
    return out_p.reshape(N, D_OUT)


# speedup vs baseline: 52.6337x; 1.0696x over previous
import jax
import jax.numpy as jnp
from jax import lax
from jax.experimental import pallas as pl
from jax.experimental.pallas import tpu as pltpu
from jax.experimental.pallas import tpu_sc as plsc

N = 100000
E = 1600000
D_IN = 128
D_LAT = 32
D_OUT = 128
HALF = 16

NSUB = 16
SUB = 128
CHUNK = 512
NSUBD = CHUNK // SUB
IDXR = E // SUB
TOTCH = E // CHUNK
NCHT = -(-TOTCH // NSUB)
ZTAIL = (N // NSUB) % CHUNK
OUT_ROWS = N // NSUB

PB = 512
PROWS = N * HALF // 128
GRID = -(-PROWS // PB)

_mesh = plsc.VectorSubcoreMesh(core_axis_name="c", subcore_axis_name="s")
_sc_params = pltpu.CompilerParams(use_tc_tiling_on_sc=False)


def _zero_fill(buf, nrows):
    zero = jnp.zeros((16,), jnp.float32)

    def zb(i, carry):
        buf[i] = zero
        return carry

    lax.fori_loop(0, nrows, zb, 0)


def _zero_acc(t, zbuf, acc):
    zbase = t * OUT_ROWS
    for j in range(OUT_ROWS // CHUNK):
        pltpu.sync_copy(zbuf.at[pl.ds(0, CHUNK)],
                        acc.at[pl.ds(zbase + j * CHUNK, CHUNK)])
    if ZTAIL:
        pltpu.sync_copy(zbuf.at[pl.ds(0, ZTAIL)],
                        acc.at[pl.ds(zbase + (OUT_ROWS // CHUNK) * CHUNK,
                                     ZTAIL)])


def _write_out(c, t, acc, out_lo, out_hi):
    ob = t * OUT_ROWS
    pl.when(c == 0)(lambda: pltpu.sync_copy(acc.at[pl.ds(ob, OUT_ROWS)],
                                            out_lo.at[pl.ds(ob, OUT_ROWS)]))
    pl.when(c == 1)(lambda: pltpu.sync_copy(acc.at[pl.ds(ob, OUT_ROWS)],
                                            out_hi.at[pl.ds(ob, OUT_ROWS)]))


def _sc_conv_body(srcp, dstp, g_lo, g_hi, out_lo, out_hi,
                  srcv, dstv, rows0, rows1, acc,
                  isem0, isem1, isem2, isem3, gsem, ssem0, ssem1):
    c = lax.axis_index("c")
    t = lax.axis_index("s")
    rows = (rows0, rows1)
    isems = (isem0, isem1, isem2, isem3)
    ssems = (ssem0, ssem1)

    _zero_fill(rows0, CHUNK)
    _zero_acc(t, rows0, acc)

    for q in range(2):
        m = 16 * q + t
        pltpu.async_copy(srcp.at[pl.ds(m * NSUBD, NSUBD)], srcv.at[q],
                         isems[q])
        pltpu.async_copy(dstp.at[pl.ds(m * NSUBD, NSUBD)], dstv.at[q],
                         isems[q])
    plsc.subcore_barrier()

    def run_chunks(g_hbm):
        def chunk_iter(i, carry):
            for q in range(4):
                j = 4 * i + q
                m = 16 * j + t

                @pl.when(m < TOTCH)
                def _(j=j, m=m, q=q):
                    h = q % 2
                    if q >= 2:
                        pltpu.make_async_copy(
                            g_hbm.at[pl.ds(0, CHUNK)], rows[h],
                            ssems[h]).wait()
                    else:
                        @pl.when(j >= 2)
                        def _():
                            pltpu.make_async_copy(
                                g_hbm.at[pl.ds(0, CHUNK)], rows[h],
                                ssems[h]).wait()
                    mn = m + 32
                    qn = (q + 2) % 4

                    @pl.when(mn < TOTCH)
                    def _(mn=mn, qn=qn):
                        pltpu.async_copy(srcp.at[pl.ds(mn * NSUBD, NSUBD)],
                                         srcv.at[qn], isems[qn])
                        pltpu.async_copy(dstp.at[pl.ds(mn * NSUBD, NSUBD)],
                                         dstv.at[qn], isems[qn])
                    pltpu.make_async_copy(srcp.at[pl.ds(0, NSUBD)],
                                          srcv.at[q], isems[q]).wait()
                    pltpu.make_async_copy(dstp.at[pl.ds(0, NSUBD)],
                                          dstv.at[q], isems[q]).wait()
                    hs = [pltpu.async_copy(
                        g_hbm.at[srcv.at[q, jj]],
                        rows[h].at[pl.ds(jj * SUB, SUB)], gsem)
                        for jj in range(NSUBD)]
                    for jj in range(NSUBD):
                        hs[jj].wait()
                        pltpu.async_copy(rows[h].at[pl.ds(jj * SUB, SUB)],
                                         acc.at[dstv.at[q, jj]], ssems[h],
                                         add=True)
            return carry

        lax.fori_loop(0, NCHT // 4, chunk_iter, 0)
        pltpu.make_async_copy(g_hbm.at[pl.ds(0, CHUNK)], rows[0],
                              ssems[0]).wait()
        pltpu.make_async_copy(g_hbm.at[pl.ds(0, CHUNK)], rows[1],
                              ssems[1]).wait()

    pl.when(c == 0)(lambda: run_chunks(g_lo))
    pl.when(c == 1)(lambda: run_chunks(g_hi))
    plsc.subcore_barrier()
    _write_out(c, t, acc, out_lo, out_hi)


_sc_conv = pl.kernel(
    _sc_conv_body,
    out_type=[jax.ShapeDtypeStruct((N, HALF), jnp.float32),
              jax.ShapeDtypeStruct((N, HALF), jnp.float32)],
    mesh=_mesh,
    scratch_types=[
        pltpu.VMEM((4, NSUBD, SUB), jnp.int32),
        pltpu.VMEM((4, NSUBD, SUB), jnp.int32),
        pltpu.VMEM((CHUNK, HALF), jnp.float32),
        pltpu.VMEM((CHUNK, HALF), jnp.float32),
        pltpu.VMEM_SHARED((N, HALF), jnp.float32),
        pltpu.SemaphoreType.DMA,
        pltpu.SemaphoreType.DMA,
        pltpu.SemaphoreType.DMA,
        pltpu.SemaphoreType.DMA,
        pltpu.SemaphoreType.DMA,
        pltpu.SemaphoreType.DMA,
        pltpu.SemaphoreType.DMA,
    ],
    compiler_params=_sc_params,
)


def _sc_deg_body(dstp, out_lo, out_hi, dstv, ones, acc,
                 isem0, isem1, isem2, isem3, ssem0, ssem1):
    c = lax.axis_index("c")
    t = lax.axis_index("s")
    w = c * NSUB + t
    isems = (isem0, isem1, isem2, isem3)
    ssems = (ssem0, ssem1)

    _zero_fill(ones, CHUNK)
    _zero_acc(t, ones, acc)

    one = jnp.full((16,), 1.0, jnp.float32)

    def ob_(i, carry):
        ones[i] = one
        return carry

    lax.fori_loop(0, SUB, ob_, 0)

    for q in range(2):
        g = 32 * q + w
        pltpu.async_copy(dstp.at[pl.ds(g * NSUBD, NSUBD)], dstv.at[q],
                         isems[q])
    plsc.subcore_barrier()

    def chunk_iter(i, carry):
        for q in range(4):
            j = 4 * i + q
            g = 32 * j + w

            @pl.when(g < TOTCH)
            def _(j=j, g=g, q=q):
                h = q % 2
                if q >= 2:
                    pltpu.make_async_copy(out_lo.at[pl.ds(0, CHUNK)],
                                          ones, ssems[h]).wait()
                else:
                    @pl.when(j >= 2)
                    def _():
                        pltpu.make_async_copy(out_lo.at[pl.ds(0, CHUNK)],
                                              ones, ssems[h]).wait()
                gn = g + 64
                qn = (q + 2) % 4

                @pl.when(gn < TOTCH)
                def _(gn=gn, qn=qn):
                    pltpu.async_copy(dstp.at[pl.ds(gn * NSUBD, NSUBD)],
                                     dstv.at[qn], isems[qn])
                pltpu.make_async_copy(dstp.at[pl.ds(0, NSUBD)],
                                      dstv.at[q], isems[q]).wait()
                for jj in range(NSUBD):
                    pltpu.async_copy(ones.at[pl.ds(0, SUB)],
                                     acc.at[dstv.at[q, jj]], ssems[h],
                                     add=True)
        return carry

    lax.fori_loop(0, -(-(-(-TOTCH // 32)) // 4), chunk_iter, 0)
    pltpu.make_async_copy(out_lo.at[pl.ds(0, CHUNK)], ones, ssems[0]).wait()
    pltpu.make_async_copy(out_lo.at[pl.ds(0, CHUNK)], ones, ssems[1]).wait()
    plsc.subcore_barrier()
    _write_out(c, t, acc, out_lo, out_hi)


_sc_deg = pl.kernel(
    _sc_deg_body,
    out_type=[jax.ShapeDtypeStruct((N, HALF), jnp.float32),
              jax.ShapeDtypeStruct((N, HALF), jnp.float32)],
    mesh=_mesh,
    scratch_types=[
        pltpu.VMEM((4, NSUBD, SUB), jnp.int32),
        pltpu.VMEM((CHUNK, HALF), jnp.float32),
        pltpu.VMEM_SHARED((N, HALF), jnp.float32),
        pltpu.SemaphoreType.DMA,
        pltpu.SemaphoreType.DMA,
        pltpu.SemaphoreType.DMA,
        pltpu.SemaphoreType.DMA,
        pltpu.SemaphoreType.DMA,
        pltpu.SemaphoreType.DMA,
    ],
    compiler_params=_sc_params,
)


def _stage_a_body(xp_ref, bdl_ref, bdh_ref, bl_ref, bh_ref, dl_ref, dh_ref,
                  dinv_ref, gl_ref, gr_ref):
    deg = dl_ref[...] + dh_ref[...] + 1.0
    dinv = lax.rsqrt(deg)
    hl = jnp.dot(xp_ref[...], bdl_ref[...],
                 preferred_element_type=jnp.float32) + bl_ref[...]
    hr = jnp.dot(xp_ref[...], bdh_ref[...],
                 preferred_element_type=jnp.float32) + bh_ref[...]
    dinv_ref[...] = dinv
    gl_ref[...] = jnp.maximum(hl, 0.0) * dinv
    gr_ref[...] = jnp.maximum(hr, 0.0) * dinv


_stage_a = pl.pallas_call(
    _stage_a_body,
    grid=(GRID,),
    in_specs=[
        pl.BlockSpec((PB, 8 * D_IN), lambda i: (i, 0)),
        pl.BlockSpec((8 * D_IN, 128), lambda i: (0, 0)),
        pl.BlockSpec((8 * D_IN, 128), lambda i: (0, 0)),
        pl.BlockSpec((1, 128), lambda i: (0, 0)),
        pl.BlockSpec((1, 128), lambda i: (0, 0)),
        pl.BlockSpec((PB, 128), lambda i: (i, 0)),
        pl.BlockSpec((PB, 128), lambda i: (i, 0)),
    ],
    out_specs=[
        pl.BlockSpec((PB, 128), lambda i: (i, 0)),
        pl.BlockSpec((PB, 128), lambda i: (i, 0)),
        pl.BlockSpec((PB, 128), lambda i: (i, 0)),
    ],
    out_shape=[jax.ShapeDtypeStruct((PROWS, 128), jnp.float32)] * 3,
)


def _stage_b_body(al_ref, ar_ref, gl_ref, gr_ref, dinv_ref,
                  bll_ref, blh_ref, bhl_ref, bhh_ref, b1l_ref, b1h_ref,
                  ol_ref, or_ref):
    dinv = dinv_ref[...]
    sl = (al_ref[...] + gl_ref[...]) * dinv
    sr = (ar_ref[...] + gr_ref[...]) * dinv
    hl = (jnp.dot(sl, bll_ref[...], preferred_element_type=jnp.float32)
          + jnp.dot(sr, bhl_ref[...], preferred_element_type=jnp.float32)
          + b1l_ref[...])
    hr = (jnp.dot(sl, blh_ref[...], preferred_element_type=jnp.float32)
          + jnp.dot(sr, bhh_ref[...], preferred_element_type=jnp.float32)
          + b1h_ref[...])
    ol_ref[...] = jnp.maximum(hl, 0.0) * dinv
    or_ref[...] = jnp.maximum(hr, 0.0) * dinv


_stage_b = pl.pallas_call(
    _stage_b_body,
    grid=(GRID,),
    in_specs=[pl.BlockSpec((PB, 128), lambda i: (i, 0))] * 5 + [
        pl.BlockSpec((128, 128), lambda i: (0, 0)),
        pl.BlockSpec((128, 128), lambda i: (0, 0)),
        pl.BlockSpec((128, 128), lambda i: (0, 0)),
        pl.BlockSpec((128, 128), lambda i: (0, 0)),
        pl.BlockSpec((1, 128), lambda i: (0, 0)),
        pl.BlockSpec((1, 128), lambda i: (0, 0)),
    ],
    out_specs=[
        pl.BlockSpec((PB, 128), lambda i: (i, 0)),
        pl.BlockSpec((PB, 128), lambda i: (i, 0)),
    ],
    out_shape=[jax.ShapeDtypeStruct((PROWS, 128), jnp.float32)] * 2,
)


def _stage_c_body(al_ref, ar_ref, gl_ref, gr_ref, dinv_ref,
                  bdl_ref, bdh_ref, b2_ref, o_ref):
    dinv = dinv_ref[...]
    sl = (al_ref[...] + gl_ref[...]) * dinv
    sr = (ar_ref[...] + gr_ref[...]) * dinv
    o_ref[...] = (jnp.dot(sl, bdl_ref[...],
                          preferred_element_type=jnp.float32)
                  + jnp.dot(sr, bdh_ref[...],
                            preferred_element_type=jnp.float32)
                  + b2_ref[...])


_stage_c = pl.pallas_call(
    _stage_c_body,
    grid=(GRID,),
    in_specs=[pl.BlockSpec((PB, 128), lambda i: (i, 0))] * 5 + [
        pl.BlockSpec((128, 8 * D_OUT), lambda i: (0, 0)),
        pl.BlockSpec((128, 8 * D_OUT), lambda i: (0, 0)),
        pl.BlockSpec((1, 8 * D_OUT), lambda i: (0, 0)),
    ],
    out_specs=pl.BlockSpec((PB, 8 * D_OUT), lambda i: (i, 0)),
    out_shape=jax.ShapeDtypeStruct((PROWS, 8 * D_OUT), jnp.float32),
)


def _bd(w):
    return jnp.kron(jnp.eye(8, dtype=w.dtype), w)


def kernel(x, edge_index, fc_W, fc_b, W1, b1, W2, b2):
    srcp = edge_index[0].reshape(IDXR, SUB)
    dstp = edge_index[1].reshape(IDXR, SUB)
    xp = x.reshape(PROWS, 8 * D_IN)

    deg_lo, deg_hi = _sc_deg(dstp)
    dinv_p, g1l_p, g1r_p = _stage_a(
        xp, _bd(fc_W[:, :HALF]), _bd(fc_W[:, HALF:]),
        jnp.tile(fc_b[:HALF], 8).reshape(1, 128),
        jnp.tile(fc_b[HALF:], 8).reshape(1, 128),
        deg_lo.reshape(PROWS, 128), deg_hi.reshape(PROWS, 128))

    a1l, a1r = _sc_conv(srcp, dstp,
                        g1l_p.reshape(N, HALF), g1r_p.reshape(N, HALF))
    g2l_p, g2r_p = _stage_b(
        a1l.reshape(PROWS, 128), a1r.reshape(PROWS, 128), g1l_p, g1r_p,
        dinv_p,
        _bd(W1[:HALF, :HALF]), _bd(W1[:HALF, HALF:]),
        _bd(W1[HALF:, :HALF]), _bd(W1[HALF:, HALF:]),
        jnp.tile(b1[:HALF], 8).reshape(1, 128),
        jnp.tile(b1[HALF:], 8).reshape(1, 128))

    a2l, a2r = _sc_conv(srcp, dstp,
                        g2l_p.reshape(N, HALF), g2r_p.reshape(N, HALF))
    out_p = _stage_c(
        a2l.reshape(PROWS, 128), a2r.reshape(PROWS, 128), g2l_p, g2r_p,
        dinv_p,
        _bd(W2[:HALF, :]), _bd(W2[HALF:, :]),
        jnp.tile(b2, 8).reshape(1, 8 * D_OUT))
    return out_p.reshape(N, D_OUT)


# speedup vs baseline: 52.6801x; 1.0009x over previous
import jax
import jax.numpy as jnp
from jax import lax
from jax.experimental import pallas as pl
from jax.experimental.pallas import tpu as pltpu
from jax.experimental.pallas import tpu_sc as plsc

N = 100000
E = 1600000
D_IN = 128
D_LAT = 32
D_OUT = 128
HALF = 16

NSUB = 16
CHUNK = 512
TOTCH = E // CHUNK
NCHT = -(-TOTCH // NSUB)
ZTAIL = (N // NSUB) % CHUNK
OUT_ROWS = N // NSUB

PB = 512
PROWS = N * HALF // 128
GRID = -(-PROWS // PB)

_mesh = plsc.VectorSubcoreMesh(core_axis_name="c", subcore_axis_name="s")
_sc_params = pltpu.CompilerParams(use_tc_tiling_on_sc=False)


def _zero_fill(buf, nrows):
    zero = jnp.zeros((16,), jnp.float32)

    def zb(i, carry):
        buf[i] = zero
        return carry

    lax.fori_loop(0, nrows, zb, 0)


def _zero_acc(t, zbuf, acc):
    zbase = t * OUT_ROWS
    for j in range(OUT_ROWS // CHUNK):
        pltpu.sync_copy(zbuf.at[pl.ds(0, CHUNK)],
                        acc.at[pl.ds(zbase + j * CHUNK, CHUNK)])
    if ZTAIL:
        pltpu.sync_copy(zbuf.at[pl.ds(0, ZTAIL)],
                        acc.at[pl.ds(zbase + (OUT_ROWS // CHUNK) * CHUNK,
                                     ZTAIL)])


def _write_out(c, t, acc, out_lo, out_hi):
    ob = t * OUT_ROWS
    pl.when(c == 0)(lambda: pltpu.sync_copy(acc.at[pl.ds(ob, OUT_ROWS)],
                                            out_lo.at[pl.ds(ob, OUT_ROWS)]))
    pl.when(c == 1)(lambda: pltpu.sync_copy(acc.at[pl.ds(ob, OUT_ROWS)],
                                            out_hi.at[pl.ds(ob, OUT_ROWS)]))


def _sc_conv_body(srcp, dstp, g_lo, g_hi, out_lo, out_hi,
                  srcv, dstv, rows0, rows1, acc,
                  isem0, isem1, isem2, isem3, gsem, ssem0, ssem1):
    c = lax.axis_index("c")
    t = lax.axis_index("s")
    rows = (rows0, rows1)
    isems = (isem0, isem1, isem2, isem3)
    ssems = (ssem0, ssem1)

    _zero_fill(rows0, CHUNK)
    _zero_acc(t, rows0, acc)

    for q in range(2):
        m = 16 * q + t
        pltpu.async_copy(srcp.at[m], srcv.at[q], isems[q])
        pltpu.async_copy(dstp.at[m], dstv.at[q], isems[q])
    plsc.subcore_barrier()

    def run_chunks(g_hbm):
        def chunk_iter(i, carry):
            for q in range(4):
                j = 4 * i + q
                m = 16 * j + t

                @pl.when(m < TOTCH)
                def _(j=j, m=m, q=q):
                    h = q % 2
                    if q >= 2:
                        pltpu.make_async_copy(
                            g_hbm.at[pl.ds(0, CHUNK)], rows[h],
                            ssems[h]).wait()
                    else:
                        @pl.when(j >= 2)
                        def _():
                            pltpu.make_async_copy(
                                g_hbm.at[pl.ds(0, CHUNK)], rows[h],
                                ssems[h]).wait()
                    mn = m + 32
                    qn = (q + 2) % 4

                    @pl.when(mn < TOTCH)
                    def _(mn=mn, qn=qn):
                        pltpu.async_copy(srcp.at[mn], srcv.at[qn],
                                         isems[qn])
                        pltpu.async_copy(dstp.at[mn], dstv.at[qn],
                                         isems[qn])
                    pltpu.make_async_copy(srcp.at[0], srcv.at[q],
                                          isems[q]).wait()
                    pltpu.make_async_copy(dstp.at[0], dstv.at[q],
                                          isems[q]).wait()
                    pltpu.async_copy(g_hbm.at[srcv.at[q]], rows[h],
                                     gsem).wait()
                    pltpu.async_copy(rows[h], acc.at[dstv.at[q]], ssems[h],
                                     add=True)
            return carry

        lax.fori_loop(0, NCHT // 4, chunk_iter, 0)
        pltpu.make_async_copy(g_hbm.at[pl.ds(0, CHUNK)], rows[0],
                              ssems[0]).wait()
        pltpu.make_async_copy(g_hbm.at[pl.ds(0, CHUNK)], rows[1],
                              ssems[1]).wait()

    pl.when(c == 0)(lambda: run_chunks(g_lo))
    pl.when(c == 1)(lambda: run_chunks(g_hi))
    plsc.subcore_barrier()
    _write_out(c, t, acc, out_lo, out_hi)


_sc_conv = pl.kernel(
    _sc_conv_body,
    out_type=[jax.ShapeDtypeStruct((N, HALF), jnp.float32),
              jax.ShapeDtypeStruct((N, HALF), jnp.float32)],
    mesh=_mesh,
    scratch_types=[
        pltpu.VMEM((4, CHUNK), jnp.int32),
        pltpu.VMEM((4, CHUNK), jnp.int32),
        pltpu.VMEM((CHUNK, HALF), jnp.float32),
        pltpu.VMEM((CHUNK, HALF), jnp.float32),
        pltpu.VMEM_SHARED((N, HALF), jnp.float32),
        pltpu.SemaphoreType.DMA,
        pltpu.SemaphoreType.DMA,
        pltpu.SemaphoreType.DMA,
        pltpu.SemaphoreType.DMA,
        pltpu.SemaphoreType.DMA,
        pltpu.SemaphoreType.DMA,
        pltpu.SemaphoreType.DMA,
    ],
    compiler_params=_sc_params,
)


def _sc_deg_body(dstp, out_lo, out_hi, dstv, ones, acc,
                 isem0, isem1, isem2, isem3, ssem0, ssem1):
    c = lax.axis_index("c")
    t = lax.axis_index("s")
    w = c * NSUB + t
    isems = (isem0, isem1, isem2, isem3)
    ssems = (ssem0, ssem1)

    _zero_fill(ones, CHUNK)
    _zero_acc(t, ones, acc)

    one = jnp.full((16,), 1.0, jnp.float32)

    def ob_(i, carry):
        ones[i] = one
        return carry

    lax.fori_loop(0, CHUNK, ob_, 0)

    for q in range(2):
        g = 32 * q + w
        pltpu.async_copy(dstp.at[g], dstv.at[q], isems[q])
    plsc.subcore_barrier()

    def chunk_iter(i, carry):
        for q in range(4):
            j = 4 * i + q
            g = 32 * j + w

            @pl.when(g < TOTCH)
            def _(j=j, g=g, q=q):
                h = q % 2
                if q >= 2:
                    pltpu.make_async_copy(out_lo.at[pl.ds(0, CHUNK)],
                                          ones, ssems[h]).wait()
                else:
                    @pl.when(j >= 2)
                    def _():
                        pltpu.make_async_copy(out_lo.at[pl.ds(0, CHUNK)],
                                              ones, ssems[h]).wait()
                gn = g + 64
                qn = (q + 2) % 4

                @pl.when(gn < TOTCH)
                def _(gn=gn, qn=qn):
                    pltpu.async_copy(dstp.at[gn], dstv.at[qn], isems[qn])
                pltpu.make_async_copy(dstp.at[0], dstv.at[q],
                                      isems[q]).wait()
                pltpu.async_copy(ones, acc.at[dstv.at[q]], ssems[h],
                                 add=True)
        return carry

    lax.fori_loop(0, -(-(-(-TOTCH // 32)) // 4), chunk_iter, 0)
    pltpu.make_async_copy(out_lo.at[pl.ds(0, CHUNK)], ones, ssems[0]).wait()
    pltpu.make_async_copy(out_lo.at[pl.ds(0, CHUNK)], ones, ssems[1]).wait()
    plsc.subcore_barrier()
    _write_out(c, t, acc, out_lo, out_hi)


_sc_deg = pl.kernel(
    _sc_deg_body,
    out_type=[jax.ShapeDtypeStruct((N, HALF), jnp.float32),
              jax.ShapeDtypeStruct((N, HALF), jnp.float32)],
    mesh=_mesh,
    scratch_types=[
        pltpu.VMEM((4, CHUNK), jnp.int32),
        pltpu.VMEM((CHUNK, HALF), jnp.float32),
        pltpu.VMEM_SHARED((N, HALF), jnp.float32),
        pltpu.SemaphoreType.DMA,
        pltpu.SemaphoreType.DMA,
        pltpu.SemaphoreType.DMA,
        pltpu.SemaphoreType.DMA,
        pltpu.SemaphoreType.DMA,
        pltpu.SemaphoreType.DMA,
    ],
    compiler_params=_sc_params,
)


def _stage_a_body(xp_ref, bdl_ref, bdh_ref, bl_ref, bh_ref, dl_ref, dh_ref,
                  dinv_ref, gl_ref, gr_ref):
    deg = dl_ref[...] + dh_ref[...] + 1.0
    dinv = lax.rsqrt(deg)
    hl = jnp.dot(xp_ref[...], bdl_ref[...],
                 preferred_element_type=jnp.float32) + bl_ref[...]
    hr = jnp.dot(xp_ref[...], bdh_ref[...],
                 preferred_element_type=jnp.float32) + bh_ref[...]
    dinv_ref[...] = dinv
    gl_ref[...] = jnp.maximum(hl, 0.0) * dinv
    gr_ref[...] = jnp.maximum(hr, 0.0) * dinv


_stage_a = pl.pallas_call(
    _stage_a_body,
    grid=(GRID,),
    in_specs=[
        pl.BlockSpec((PB, 8 * D_IN), lambda i: (i, 0)),
        pl.BlockSpec((8 * D_IN, 128), lambda i: (0, 0)),
        pl.BlockSpec((8 * D_IN, 128), lambda i: (0, 0)),
        pl.BlockSpec((1, 128), lambda i: (0, 0)),
        pl.BlockSpec((1, 128), lambda i: (0, 0)),
        pl.BlockSpec((PB, 128), lambda i: (i, 0)),
        pl.BlockSpec((PB, 128), lambda i: (i, 0)),
    ],
    out_specs=[
        pl.BlockSpec((PB, 128), lambda i: (i, 0)),
        pl.BlockSpec((PB, 128), lambda i: (i, 0)),
        pl.BlockSpec((PB, 128), lambda i: (i, 0)),
    ],
    out_shape=[jax.ShapeDtypeStruct((PROWS, 128), jnp.float32)] * 3,
)


def _stage_b_body(al_ref, ar_ref, gl_ref, gr_ref, dinv_ref,
                  bll_ref, blh_ref, bhl_ref, bhh_ref, b1l_ref, b1h_ref,
                  ol_ref, or_ref):
    dinv = dinv_ref[...]
    sl = (al_ref[...] + gl_ref[...]) * dinv
    sr = (ar_ref[...] + gr_ref[...]) * dinv
    hl = (jnp.dot(sl, bll_ref[...], preferred_element_type=jnp.float32)
          + jnp.dot(sr, bhl_ref[...], preferred_element_type=jnp.float32)
          + b1l_ref[...])
    hr = (jnp.dot(sl, blh_ref[...], preferred_element_type=jnp.float32)
          + jnp.dot(sr, bhh_ref[...], preferred_element_type=jnp.float32)
          + b1h_ref[...])
    ol_ref[...] = jnp.maximum(hl, 0.0) * dinv
    or_ref[...] = jnp.maximum(hr, 0.0) * dinv


_stage_b = pl.pallas_call(
    _stage_b_body,
    grid=(GRID,),
    in_specs=[pl.BlockSpec((PB, 128), lambda i: (i, 0))] * 5 + [
        pl.BlockSpec((128, 128), lambda i: (0, 0)),
        pl.BlockSpec((128, 128), lambda i: (0, 0)),
        pl.BlockSpec((128, 128), lambda i: (0, 0)),
        pl.BlockSpec((128, 128), lambda i: (0, 0)),
        pl.BlockSpec((1, 128), lambda i: (0, 0)),
        pl.BlockSpec((1, 128), lambda i: (0, 0)),
    ],
    out_specs=[
        pl.BlockSpec((PB, 128), lambda i: (i, 0)),
        pl.BlockSpec((PB, 128), lambda i: (i, 0)),
    ],
    out_shape=[jax.ShapeDtypeStruct((PROWS, 128), jnp.float32)] * 2,
)


def _stage_c_body(al_ref, ar_ref, gl_ref, gr_ref, dinv_ref,
                  bdl_ref, bdh_ref, b2_ref, o_ref):
    dinv = dinv_ref[...]
    sl = (al_ref[...] + gl_ref[...]) * dinv
    sr = (ar_ref[...] + gr_ref[...]) * dinv
    o_ref[...] = (jnp.dot(sl, bdl_ref[...],
                          preferred_element_type=jnp.float32)
                  + jnp.dot(sr, bdh_ref[...],
                            preferred_element_type=jnp.float32)
                  + b2_ref[...])


_stage_c = pl.pallas_call(
    _stage_c_body,
    grid=(GRID,),
    in_specs=[pl.BlockSpec((PB, 128), lambda i: (i, 0))] * 5 + [
        pl.BlockSpec((128, 8 * D_OUT), lambda i: (0, 0)),
        pl.BlockSpec((128, 8 * D_OUT), lambda i: (0, 0)),
        pl.BlockSpec((1, 8 * D_OUT), lambda i: (0, 0)),
    ],
    out_specs=pl.BlockSpec((PB, 8 * D_OUT), lambda i: (i, 0)),
    out_shape=jax.ShapeDtypeStruct((PROWS, 8 * D_OUT), jnp.float32),
)


def _bd(w):
    return jnp.kron(jnp.eye(8, dtype=w.dtype), w)


def kernel(x, edge_index, fc_W, fc_b, W1, b1, W2, b2):
    srcp = edge_index[0].reshape(TOTCH, CHUNK)
    dstp = edge_index[1].reshape(TOTCH, CHUNK)
    xp = x.reshape(PROWS, 8 * D_IN)

    deg_lo, deg_hi = _sc_deg(dstp)
    dinv_p, g1l_p, g1r_p = _stage_a(
        xp, _bd(fc_W[:, :HALF]), _bd(fc_W[:, HALF:]),
        jnp.tile(fc_b[:HALF], 8).reshape(1, 128),
        jnp.tile(fc_b[HALF:], 8).reshape(1, 128),
        deg_lo.reshape(PROWS, 128), deg_hi.reshape(PROWS, 128))

    a1l, a1r = _sc_conv(srcp, dstp,
                        g1l_p.reshape(N, HALF), g1r_p.reshape(N, HALF))
    g2l_p, g2r_p = _stage_b(
        a1l.reshape(PROWS, 128), a1r.reshape(PROWS, 128), g1l_p, g1r_p,
        dinv_p,
        _bd(W1[:HALF, :HALF]), _bd(W1[:HALF, HALF:]),
        _bd(W1[HALF:, :HALF]), _bd(W1[HALF:, HALF:]),
        jnp.tile(b1[:HALF], 8).reshape(1, 128),
        jnp.tile(b1[HALF:], 8).reshape(1, 128))

    a2l, a2r = _sc_conv(srcp, dstp,
                        g2l_p.reshape(N, HALF), g2r_p.reshape(N, HALF))
    out_p = _stage_c(
        a2l.reshape(PROWS, 128), a2r.reshape(PROWS, 128), g2l_p, g2r_p,
        dinv_p,
        _bd(W2[:HALF, :]), _bd(W2[HALF:, :]),
        jnp.tile(b2, 8).reshape(1, 8 * D_OUT))
    return out_p.reshape(N, D_OUT)
